# Initial kernel scaffold; baseline (speedup 1.0000x reference)
import jax
import jax.numpy as jnp
from jax.experimental import pallas as pl


def kernel(x, edge_index, rel_type, norm, params):
    raise NotImplementedError("write your pallas kernel here")


# speedup vs baseline: 21.7322x; 21.7322x over previous
import functools

import jax
import jax.numpy as jnp
from jax import lax
from jax.experimental import pallas as pl
from jax.experimental.pallas import tpu as pltpu
from jax.experimental.pallas import tpu_sc as plsc

N0 = 50000
E0 = 1600000
D0 = 16
NREL = 3
NW = 32
EPW = E0 // NW
CHUNK = 80
NCHUNK = EPW // CHUNK
HALF = N0 // 2


def _edge_body(pcat, gidx, dstv, nrm, zeros, out, acc,
               idx0, idx1, dst0, dst1, nrm0, nrm1, rows0, rows1,
               gsem0, gsem1, msem0, msem1):
    c = lax.axis_index("c")
    s = lax.axis_index("s")
    w = c * 16 + s
    base = w * EPW

    @pl.when(s == 0)
    def _():
        pltpu.sync_copy(zeros, acc)
    plsc.subcore_barrier()

    idx = (idx0, idx1)
    dstb = (dst0, dst1)
    nrmb = (nrm0, nrm1)
    rows = (rows0, rows1)
    gsem = (gsem0, gsem1)
    msem = (msem0, msem1)

    def meta_load_sync(chunk_i, b):
        off = base + chunk_i * CHUNK
        pltpu.sync_copy(gidx.at[pl.ds(off, CHUNK)], idx[b])
        pltpu.sync_copy(dstv.at[pl.ds(off, CHUNK)], dstb[b])
        pltpu.sync_copy(nrm.at[pl.ds(off, CHUNK)], nrmb[b])

    def meta_load(chunk_i, b):
        off = base + chunk_i * CHUNK
        pltpu.async_copy(gidx.at[pl.ds(off, CHUNK)], idx[b], msem[b])
        pltpu.async_copy(dstv.at[pl.ds(off, CHUNK)], dstb[b], msem[b])
        pltpu.async_copy(nrm.at[pl.ds(off, CHUNK)], nrmb[b], msem[b])

    def meta_wait(b):
        pltpu.make_async_copy(gidx.at[pl.ds(0, CHUNK)], idx[b], msem[b]).wait()
        pltpu.make_async_copy(dstv.at[pl.ds(0, CHUNK)], dstb[b], msem[b]).wait()
        pltpu.make_async_copy(nrm.at[pl.ds(0, CHUNK)], nrmb[b], msem[b]).wait()

    def gather_issue(b):
        pltpu.async_copy(pcat.at[idx[b]], rows[b], gsem[b])

    def gather_wait(b):
        pltpu.make_async_copy(pcat.at[idx[b]], rows[b], gsem[b]).wait()

    def process(b):
        gather_wait(b)

        def sbody(j, carry):
            nrm16 = nrmb[b][pl.ds(j * 16, 16)]
            for e16 in range(16):
                e = j * 16 + e16
                rows[b][e] = rows[b][e] * nrm16[e16]
            return carry

        lax.fori_loop(0, CHUNK // 16, sbody, 0)
        pltpu.sync_copy(rows[b], acc.at[dstb[b]], add=True)

    meta_load_sync(0, 0)
    gather_issue(0)
    meta_load(1, 1)

    def outer(i, carry):
        for b in (0, 1):
            cc = 2 * i + b
            meta_wait(1 - b)
            gather_issue(1 - b)
            process(b)

            @pl.when(cc + 2 < NCHUNK)
            def _():
                meta_load(cc + 2, b)
        return carry

    lax.fori_loop(0, (NCHUNK - 1) // 2, outer, 0)
    process(0)

    plsc.subcore_barrier()

    @pl.when(s == 0)
    def _():
        pltpu.sync_copy(acc.at[pl.ds(0, HALF)], out.at[pl.ds(c * N0, HALF)])

    @pl.when(s == 8)
    def _():
        pltpu.sync_copy(acc.at[pl.ds(HALF, HALF)],
                        out.at[pl.ds(c * N0 + HALF, HALF)])


@functools.cache
def _edge_pass_fn():
    return pl.kernel(
        _edge_body,
        out_type=jax.ShapeDtypeStruct((2 * N0, D0), jnp.float32),
        mesh=plsc.VectorSubcoreMesh(core_axis_name="c", subcore_axis_name="s",
                                    num_cores=2, num_subcores=16),
        compiler_params=pltpu.CompilerParams(use_tc_tiling_on_sc=False),
        scratch_types=[
            pltpu.VMEM_SHARED((N0, D0), jnp.float32),
            pltpu.VMEM((CHUNK,), jnp.int32),
            pltpu.VMEM((CHUNK,), jnp.int32),
            pltpu.VMEM((CHUNK,), jnp.int32),
            pltpu.VMEM((CHUNK,), jnp.int32),
            pltpu.VMEM((CHUNK,), jnp.float32),
            pltpu.VMEM((CHUNK,), jnp.float32),
            pltpu.VMEM((CHUNK, D0), jnp.float32),
            pltpu.VMEM((CHUNK, D0), jnp.float32),
            pltpu.SemaphoreType.DMA,
            pltpu.SemaphoreType.DMA,
            pltpu.SemaphoreType.DMA,
            pltpu.SemaphoreType.DMA,
        ],
    )


def _edge_pass(*args):
    return _edge_pass_fn()(*args)


R8 = N0 // 8


def _bn_packed(y, G, g, b):
    mB = jnp.dot(jnp.mean(y, axis=0, keepdims=True), G,
                 preferred_element_type=jnp.float32)
    sB = jnp.dot(jnp.mean(y * y, axis=0, keepdims=True), G,
                 preferred_element_type=jnp.float32)
    vB = sB - mB * mB
    return g * (y - mB) / jnp.sqrt(vB + 1e-5) + b


def _embed_proj_body(x_ref, embW_bd, embb, embg, embbeta, G, rgcnW_bd,
                     pcat_ref):
    y = jnp.dot(x_ref[...], embW_bd[...],
                preferred_element_type=jnp.float32) + embb[...]
    h0 = jnp.maximum(_bn_packed(y, G[...], embg[...], embbeta[...]), 0.0)
    for r in range(NREL):
        pcat_ref[r] = jnp.dot(h0, rgcnW_bd[r],
                              preferred_element_type=jnp.float32)


def _gru_gates(xin, W_bd, b3):
    g0 = jnp.dot(xin, W_bd[0], preferred_element_type=jnp.float32) + b3[0]
    g1 = jnp.dot(xin, W_bd[1], preferred_element_type=jnp.float32) + b3[1]
    g2 = jnp.dot(xin, W_bd[2], preferred_element_type=jnp.float32) + b3[2]
    return g0, g1, g2


def _gru1_body(S, Wih_bd, bih3, bhh3, rgcnW_bd, h_ref, pcat_ref):
    swh = S[0:R8, :] + S[R8:2 * R8, :]
    ir, iz, i_n = _gru_gates(swh, Wih_bd[...], bih3[...])
    rg = jax.nn.sigmoid(ir + bhh3[0])
    zg = jax.nn.sigmoid(iz + bhh3[1])
    ng = jnp.tanh(i_n + rg * bhh3[2])
    h = (1.0 - zg) * ng
    h_ref[...] = h
    for r in range(NREL):
        pcat_ref[r] = jnp.dot(h, rgcnW_bd[r],
                              preferred_element_type=jnp.float32)


def _final_body(S, h1, Wih_bd, bih3, Whh_bd, bhh3, G, kbng, kbnb,
                taW1_bd, tab1, tag, tabeta, taW2_bd, tab2,
                tbW1_bd, tbb1, tbg, tbbeta, tbW2_bd, tbb2,
                xa_ref, xb_ref):
    swh = S[0:R8, :] + S[R8:2 * R8, :]
    hp = h1[...]
    ir, iz, i_n = _gru_gates(swh, Wih_bd[...], bih3[...])
    hr, hz, h_n = _gru_gates(hp, Whh_bd[...], bhh3[...])
    rg = jax.nn.sigmoid(ir + hr)
    zg = jax.nn.sigmoid(iz + hz)
    ng = jnp.tanh(i_n + rg * h_n)
    h2 = (1.0 - zg) * ng + zg * hp
    Gm = G[...]
    hf = _bn_packed(h2, Gm, kbng[...], kbnb[...])
    ya = jnp.maximum(_bn_packed(
        jnp.dot(hf, taW1_bd[...], preferred_element_type=jnp.float32)
        + tab1[...], Gm, tag[...], tabeta[...]), 0.0)
    xa_ref[...] = jnp.dot(ya, taW2_bd[...],
                          preferred_element_type=jnp.float32) + tab2[...]
    yb = jnp.maximum(_bn_packed(
        jnp.dot(hf, tbW1_bd[...], preferred_element_type=jnp.float32)
        + tbb1[...], Gm, tbg[...], tbbeta[...]), 0.0)
    xb_ref[...] = jnp.dot(yb, tbW2_bd[...],
                          preferred_element_type=jnp.float32) + tbb2[...]


def kernel(x, edge_index, rel_type, norm, params):
    p = params
    src = edge_index[0]
    dst = edge_index[1]
    gidx = rel_type * jnp.int32(N0) + src
    zeros = jnp.zeros((N0, D0), jnp.float32)

    eye8 = jnp.eye(8, dtype=jnp.float32)
    bd = lambda W: jnp.kron(eye8, W)
    bd3 = lambda W3: jnp.stack([bd(W3[r]) for r in range(NREL)])
    tile8 = lambda v: jnp.tile(v, 8)
    G = jnp.kron(jnp.ones((8, 8), jnp.float32) / 8.0,
                 jnp.eye(D0, dtype=jnp.float32))

    embW_bd = bd(p['emb_W'].T)
    rgcnW_bd = bd3(p['rgcn_W'])
    Wih_bd = bd3(jnp.transpose(p['gru_Wih'].reshape(NREL, D0, D0), (0, 2, 1)))
    Whh_bd = bd3(jnp.transpose(p['gru_Whh'].reshape(NREL, D0, D0), (0, 2, 1)))
    bih3 = jnp.tile(p['gru_bih'].reshape(NREL, D0), (1, 8))
    bhh3 = jnp.tile(p['gru_bhh'].reshape(NREL, D0), (1, 8))

    pcat1 = pl.pallas_call(
        _embed_proj_body,
        out_shape=jax.ShapeDtypeStruct((NREL, R8, 128), jnp.float32),
    )(x.reshape(R8, 128), embW_bd, tile8(p['emb_b']), tile8(p['emb_g']),
      tile8(p['emb_beta']), G, rgcnW_bd)

    S1 = _edge_pass(pcat1.reshape(NREL * N0, D0), gidx, dst, norm, zeros)

    h1, pcat2 = pl.pallas_call(
        _gru1_body,
        out_shape=(jax.ShapeDtypeStruct((R8, 128), jnp.float32),
                   jax.ShapeDtypeStruct((NREL, R8, 128), jnp.float32)),
    )(S1.reshape(2 * R8, 128), Wih_bd, bih3, bhh3, rgcnW_bd)

    S2 = _edge_pass(pcat2.reshape(NREL * N0, D0), gidx, dst, norm, zeros)

    xa, xb = pl.pallas_call(
        _final_body,
        out_shape=(jax.ShapeDtypeStruct((R8, 16), jnp.float32),
                   jax.ShapeDtypeStruct((R8, 128), jnp.float32)),
    )(S2.reshape(2 * R8, 128), h1, Wih_bd, bih3, Whh_bd, bhh3, G,
      tile8(p['kbn_g']), tile8(p['kbn_b']),
      bd(p['ta_W1'].T), tile8(p['ta_b1']), tile8(p['ta_g']),
      tile8(p['ta_beta']), bd(p['ta_W2'].T), tile8(p['ta_b2']),
      bd(p['tb_W1'].T), tile8(p['tb_b1']), tile8(p['tb_g']),
      tile8(p['tb_beta']), bd(p['tb_W2'].T), tile8(p['tb_b2']))
    return (xa.reshape(N0, 2), xb.reshape(N0, 16))


# speedup vs baseline: 27.4500x; 1.2631x over previous
import functools

import jax
import jax.numpy as jnp
from jax import lax
from jax.experimental import pallas as pl
from jax.experimental.pallas import tpu as pltpu
from jax.experimental.pallas import tpu_sc as plsc

N0 = 50000
E0 = 1600000
D0 = 16
NREL = 3
NW = 32
CHUNK = 128
NCH = 391
EPW = NCH * CHUNK
EPAD = NW * EPW
HALF = N0 // 2


def _edge_body(pcat, meta, zeros, out, acc,
               meta0, meta1, dstix0, dstix1, rows0, rows1,
               gsem0, gsem1, msem0, msem1, ssem0, ssem1):
    c = lax.axis_index("c")
    s = lax.axis_index("s")
    w = c * 16 + s
    base = w * NCH

    @pl.when(s == 0)
    def _():
        pltpu.sync_copy(zeros, acc)
    plsc.subcore_barrier()

    metab = (meta0, meta1)
    dstix = (dstix0, dstix1)
    rows = (rows0, rows1)
    gsem = (gsem0, gsem1)
    msem = (msem0, msem1)
    ssem = (ssem0, ssem1)

    def meta_slice(chunk_i):
        return meta.at[pl.ds(3 * (base + chunk_i), 3)]

    def meta_load_sync(chunk_i, b):
        pltpu.sync_copy(meta_slice(chunk_i), metab[b])

    def meta_load(chunk_i, b):
        pltpu.async_copy(meta_slice(chunk_i), metab[b], msem[b])

    def meta_wait(b):
        pltpu.make_async_copy(meta_slice(0), metab[b], msem[b]).wait()

    def gather_issue(b):
        pltpu.async_copy(pcat.at[metab[b].at[0]], rows[b], gsem[b])

    def gather_wait(b):
        pltpu.make_async_copy(pcat.at[metab[b].at[0]], rows[b], gsem[b]).wait()

    def scatter_issue(b):
        pltpu.async_copy(rows[b], acc.at[dstix[b]], ssem[b], add=True)

    def scatter_wait(b):
        pltpu.make_async_copy(rows[b], acc.at[dstix[b]], ssem[b]).wait()

    def scale(b):
        def sbody(j, carry):
            nrm16 = plsc.bitcast(metab[b][2, pl.ds(j * 16, 16)], jnp.float32)
            dstix[b][pl.ds(j * 16, 16)] = metab[b][1, pl.ds(j * 16, 16)]
            for e16 in range(16):
                e = j * 16 + e16
                rows[b][e] = rows[b][e] * nrm16[e16]
            return carry

        lax.fori_loop(0, CHUNK // 16, sbody, 0)

    meta_load_sync(0, 0)
    gather_issue(0)
    meta_load(1, 1)

    def outer(i, carry):
        for b in (0, 1):
            cc = 2 * i + b
            meta_wait(1 - b)

            @pl.when(cc > 0)
            def _():
                scatter_wait(1 - b)

            gather_issue(1 - b)
            gather_wait(b)
            scale(b)
            scatter_issue(b)

            @pl.when(cc + 2 < NCH)
            def _():
                meta_load(cc + 2, b)
        return carry

    lax.fori_loop(0, (NCH - 1) // 2, outer, 0)
    scatter_wait(1)
    gather_wait(0)
    scale(0)
    pltpu.sync_copy(rows[0], acc.at[dstix[0]], add=True)

    plsc.subcore_barrier()

    @pl.when(s == 0)
    def _():
        pltpu.sync_copy(acc.at[pl.ds(0, HALF)], out.at[pl.ds(c * N0, HALF)])

    @pl.when(s == 8)
    def _():
        pltpu.sync_copy(acc.at[pl.ds(HALF, HALF)],
                        out.at[pl.ds(c * N0 + HALF, HALF)])


@functools.cache
def _edge_pass_fn():
    return pl.kernel(
        _edge_body,
        out_type=jax.ShapeDtypeStruct((2 * N0, D0), jnp.float32),
        mesh=plsc.VectorSubcoreMesh(core_axis_name="c", subcore_axis_name="s",
                                    num_cores=2, num_subcores=16),
        compiler_params=pltpu.CompilerParams(use_tc_tiling_on_sc=False,
                                             needs_layout_passes=False),
        scratch_types=[
            pltpu.VMEM_SHARED((N0, D0), jnp.float32),
            pltpu.VMEM((3, CHUNK), jnp.int32),
            pltpu.VMEM((3, CHUNK), jnp.int32),
            pltpu.VMEM((CHUNK,), jnp.int32),
            pltpu.VMEM((CHUNK,), jnp.int32),
            pltpu.VMEM((CHUNK, D0), jnp.float32),
            pltpu.VMEM((CHUNK, D0), jnp.float32),
            pltpu.SemaphoreType.DMA,
            pltpu.SemaphoreType.DMA,
            pltpu.SemaphoreType.DMA,
            pltpu.SemaphoreType.DMA,
            pltpu.SemaphoreType.DMA,
            pltpu.SemaphoreType.DMA,
        ],
    )


def _edge_pass(*args):
    return _edge_pass_fn()(*args)


R8 = N0 // 8


def _bn_packed(y, G, g, b):
    mB = jnp.dot(jnp.mean(y, axis=0, keepdims=True), G,
                 preferred_element_type=jnp.float32)
    sB = jnp.dot(jnp.mean(y * y, axis=0, keepdims=True), G,
                 preferred_element_type=jnp.float32)
    vB = sB - mB * mB
    return g * (y - mB) / jnp.sqrt(vB + 1e-5) + b


def _embed_proj_body(x_ref, embW_bd, embb, embg, embbeta, G, rgcnW_bd,
                     pcat_ref):
    y = jnp.dot(x_ref[...], embW_bd[...],
                preferred_element_type=jnp.float32) + embb[...]
    h0 = jnp.maximum(_bn_packed(y, G[...], embg[...], embbeta[...]), 0.0)
    for r in range(NREL):
        pcat_ref[r] = jnp.dot(h0, rgcnW_bd[r],
                              preferred_element_type=jnp.float32)


def _gru_gates(xin, W_bd, b3):
    g0 = jnp.dot(xin, W_bd[0], preferred_element_type=jnp.float32) + b3[0]
    g1 = jnp.dot(xin, W_bd[1], preferred_element_type=jnp.float32) + b3[1]
    g2 = jnp.dot(xin, W_bd[2], preferred_element_type=jnp.float32) + b3[2]
    return g0, g1, g2


def _gru1_body(S, Wih_bd, bih3, bhh3, rgcnW_bd, h_ref, pcat_ref):
    swh = S[0:R8, :] + S[R8:2 * R8, :]
    ir, iz, i_n = _gru_gates(swh, Wih_bd[...], bih3[...])
    rg = jax.nn.sigmoid(ir + bhh3[0])
    zg = jax.nn.sigmoid(iz + bhh3[1])
    ng = jnp.tanh(i_n + rg * bhh3[2])
    h = (1.0 - zg) * ng
    h_ref[...] = h
    for r in range(NREL):
        pcat_ref[r] = jnp.dot(h, rgcnW_bd[r],
                              preferred_element_type=jnp.float32)


def _final_body(S, h1, Wih_bd, bih3, Whh_bd, bhh3, G, kbng, kbnb,
                taW1_bd, tab1, tag, tabeta, taW2_bd, tab2,
                tbW1_bd, tbb1, tbg, tbbeta, tbW2_bd, tbb2,
                xa_ref, xb_ref):
    swh = S[0:R8, :] + S[R8:2 * R8, :]
    hp = h1[...]
    ir, iz, i_n = _gru_gates(swh, Wih_bd[...], bih3[...])
    hr, hz, h_n = _gru_gates(hp, Whh_bd[...], bhh3[...])
    rg = jax.nn.sigmoid(ir + hr)
    zg = jax.nn.sigmoid(iz + hz)
    ng = jnp.tanh(i_n + rg * h_n)
    h2 = (1.0 - zg) * ng + zg * hp
    Gm = G[...]
    hf = _bn_packed(h2, Gm, kbng[...], kbnb[...])
    ya = jnp.maximum(_bn_packed(
        jnp.dot(hf, taW1_bd[...], preferred_element_type=jnp.float32)
        + tab1[...], Gm, tag[...], tabeta[...]), 0.0)
    xa_ref[...] = jnp.dot(ya, taW2_bd[...],
                          preferred_element_type=jnp.float32) + tab2[...]
    yb = jnp.maximum(_bn_packed(
        jnp.dot(hf, tbW1_bd[...], preferred_element_type=jnp.float32)
        + tbb1[...], Gm, tbg[...], tbbeta[...]), 0.0)
    xb_ref[...] = jnp.dot(yb, tbW2_bd[...],
                          preferred_element_type=jnp.float32) + tbb2[...]


def kernel(x, edge_index, rel_type, norm, params):
    p = params
    src = edge_index[0]
    dst = edge_index[1]
    gidx = rel_type * jnp.int32(N0) + src
    zeros = jnp.zeros((N0, D0), jnp.float32)

    pad = EPAD - E0
    gidx_p = jnp.concatenate([gidx, jnp.zeros((pad,), jnp.int32)])
    dst_p = jnp.concatenate([dst, jnp.zeros((pad,), jnp.int32)])
    nrm_p = jnp.concatenate([lax.bitcast_convert_type(norm, jnp.int32),
                             jnp.zeros((pad,), jnp.int32)])
    meta = jnp.stack([gidx_p.reshape(-1, CHUNK), dst_p.reshape(-1, CHUNK),
                      nrm_p.reshape(-1, CHUNK)], axis=1).reshape(-1, CHUNK)

    eye8 = jnp.eye(8, dtype=jnp.float32)
    bd = lambda W: jnp.kron(eye8, W)
    bd3 = lambda W3: jnp.stack([bd(W3[r]) for r in range(NREL)])
    tile8 = lambda v: jnp.tile(v, 8)
    G = jnp.kron(jnp.ones((8, 8), jnp.float32) / 8.0,
                 jnp.eye(D0, dtype=jnp.float32))

    embW_bd = bd(p['emb_W'].T)
    rgcnW_bd = bd3(p['rgcn_W'])
    Wih_bd = bd3(jnp.transpose(p['gru_Wih'].reshape(NREL, D0, D0), (0, 2, 1)))
    Whh_bd = bd3(jnp.transpose(p['gru_Whh'].reshape(NREL, D0, D0), (0, 2, 1)))
    bih3 = jnp.tile(p['gru_bih'].reshape(NREL, D0), (1, 8))
    bhh3 = jnp.tile(p['gru_bhh'].reshape(NREL, D0), (1, 8))

    pcat1 = pl.pallas_call(
        _embed_proj_body,
        out_shape=jax.ShapeDtypeStruct((NREL, R8, 128), jnp.float32),
    )(x.reshape(R8, 128), embW_bd, tile8(p['emb_b']), tile8(p['emb_g']),
      tile8(p['emb_beta']), G, rgcnW_bd)

    S1 = _edge_pass(pcat1.reshape(NREL * N0, D0), meta, zeros)

    h1, pcat2 = pl.pallas_call(
        _gru1_body,
        out_shape=(jax.ShapeDtypeStruct((R8, 128), jnp.float32),
                   jax.ShapeDtypeStruct((NREL, R8, 128), jnp.float32)),
    )(S1.reshape(2 * R8, 128), Wih_bd, bih3, bhh3, rgcnW_bd)

    S2 = _edge_pass(pcat2.reshape(NREL * N0, D0), meta, zeros)

    xa, xb = pl.pallas_call(
        _final_body,
        out_shape=(jax.ShapeDtypeStruct((R8, 16), jnp.float32),
                   jax.ShapeDtypeStruct((R8, 128), jnp.float32)),
    )(S2.reshape(2 * R8, 128), h1, Wih_bd, bih3, Whh_bd, bhh3, G,
      tile8(p['kbn_g']), tile8(p['kbn_b']),
      bd(p['ta_W1'].T), tile8(p['ta_b1']), tile8(p['ta_g']),
      tile8(p['ta_beta']), bd(p['ta_W2'].T), tile8(p['ta_b2']),
      bd(p['tb_W1'].T), tile8(p['tb_b1']), tile8(p['tb_g']),
      tile8(p['tb_beta']), bd(p['tb_W2'].T), tile8(p['tb_b2']))
    return (xa.reshape(N0, 2), xb.reshape(N0, 16))
